# Initial kernel scaffold; baseline (speedup 1.0000x reference)
import jax
import jax.numpy as jnp
from jax.experimental import pallas as pl


def kernel(x, W_in, W_out):
    raise NotImplementedError("write your pallas kernel here")


# speedup vs baseline: 1.8711x; 1.8711x over previous
import functools

import jax
import jax.numpy as jnp
from jax import lax
from jax.experimental import pallas as pl
from jax.experimental.pallas import tpu as pltpu
from jax.experimental.pallas import tpu_sc as plsc

VOCAB = 1000000
DIM = 64
BATCH = 16384
HIST = 50

NW = 32
C = 128
B = BATCH * HIST
CPW = B // (NW * C)
NB = 4


def _emb_body(x_hbm, tab_hbm, out_hbm, idx_v, b0, b1, b2, b3,
              g0, g1, g2, g3, s0, s1, s2, s3):
    bufs = (b0, b1, b2, b3)
    gsems = (g0, g1, g2, g3)
    ssems = (s0, s1, s2, s3)
    wid = lax.axis_index("c") * 16 + lax.axis_index("s")
    base = wid * CPW

    pltpu.sync_copy(x_hbm.at[pl.ds(base, CPW)], idx_v)

    def start_g(j, b):
        pltpu.make_async_copy(
            tab_hbm.at[idx_v.at[j]], bufs[b], gsems[b]).start()

    def wait_g(j, b):
        pltpu.make_async_copy(
            tab_hbm.at[idx_v.at[j]], bufs[b], gsems[b]).wait()

    def start_s(j, b):
        pltpu.make_async_copy(bufs[b], out_hbm.at[base + j], ssems[b]).start()

    def wait_s(j, b):
        pltpu.make_async_copy(bufs[b], out_hbm.at[base + j], ssems[b]).wait()

    for b in range(NB):
        start_g(b, b)

    def body(i, carry):
        g = i * NB
        for b in range(NB):
            wait_g(g + b, b)
            start_s(g + b, b)
        for b in range(NB):
            wait_s(g + b, b)
            start_g(g + NB + b, b)
        return carry

    lax.fori_loop(0, CPW // NB - 1, body, 0)

    g = CPW - NB
    for b in range(NB):
        wait_g(g + b, b)
        start_s(g + b, b)
    for b in range(NB):
        wait_s(g + b, b)


@functools.partial(jax.jit, static_argnums=())
def _embed(x2d, table):
    mesh = plsc.VectorSubcoreMesh(core_axis_name="c", subcore_axis_name="s")
    f = functools.partial(
        pl.kernel,
        mesh=mesh,
        out_type=jax.ShapeDtypeStruct((B // C, C, DIM), jnp.float32),
        scratch_types=(
            [pltpu.VMEM((CPW, C), jnp.int32)]
            + [pltpu.VMEM((C, DIM), jnp.float32) for _ in range(NB)]
            + [pltpu.SemaphoreType.DMA for _ in range(2 * NB)]
        ),
        compiler_params=pltpu.CompilerParams(use_tc_tiling_on_sc=False),
    )(_emb_body)
    return f(x2d, table)


def kernel(x, W_in, W_out):
    x2d = x.reshape(B // C, C).astype(jnp.int32)
    out = _embed(x2d, W_in)
    return out.reshape(BATCH, HIST, DIM)
